# baseline (speedup 1.0000x reference)
import functools

import jax
import jax.numpy as jnp
from jax import lax
from jax.experimental import pallas as pl
from jax.experimental.pallas import tpu as pltpu
from jax.experimental.pallas import tpu_sc as plsc


def _scores_body(x_ref, w_ref, bias_ref, scores_ref, g_ref, hist_ref):
    xb = x_ref[...].astype(jnp.bfloat16).astype(jnp.float32)
    wv = w_ref[...].astype(jnp.bfloat16).astype(jnp.float32)
    s = jnp.sum(xb * wv, axis=-1)
    s = s + bias_ref[0, 0]
    scores_ref[0, 0, :] = s
    g_ref[0, 0, :] = jax.nn.sigmoid(s)

    uu = lax.bitcast_convert_type(s, jnp.uint32)
    neg = uu >= jnp.uint32(0x80000000)
    ku = uu ^ jnp.where(neg, jnp.uint32(0xFFFFFFFF), jnp.uint32(0x80000000))
    tb = (ku >> jnp.uint32(24)).astype(jnp.int32)

    @pl.when(pl.program_id(0) == 0)
    def _():
        hist_ref[0, :] = jnp.zeros((256,), jnp.int32)
    bins16 = lax.broadcasted_iota(jnp.int32, (16, 1), 0)
    for g16 in range(16):
        eqm = tb[None, :] == (bins16 + g16 * 16)
        hist_ref[0, g16 * 16:(g16 + 1) * 16] += jnp.sum(
            eqm.astype(jnp.int32), axis=1)


def _compute_scores(x2, W, bias):
    n, h = x2.shape
    rows = 2048
    grid = n // rows
    return pl.pallas_call(
        _scores_body,
        grid=(grid,),
        in_specs=[
            pl.BlockSpec((rows, h), lambda i: (i, 0)),
            pl.BlockSpec((1, h), lambda i: (0, 0)),
            pl.BlockSpec(memory_space=pltpu.SMEM),
        ],
        out_specs=[
            pl.BlockSpec((1, 1, rows), lambda i: (i, 0, 0)),
            pl.BlockSpec((1, 1, rows), lambda i: (i, 0, 0)),
            pl.BlockSpec((1, 256), lambda i: (0, 0)),
        ],
        out_shape=[
            jax.ShapeDtypeStruct((grid, 1, rows), jnp.float32),
            jax.ShapeDtypeStruct((grid, 1, rows), jnp.float32),
            jax.ShapeDtypeStruct((1, 256), jnp.int32),
        ],
        compiler_params=pltpu.CompilerParams(
            dimension_semantics=("arbitrary",),
        ),
    )(x2, W, bias)


_LANES = 16
_NT = 16
_NW = 32


def _keys16(v):
    uu = lax.bitcast_convert_type(v, jnp.uint32)
    neg = uu >= jnp.uint32(0x80000000)
    xm = jnp.where(neg, jnp.uint32(0xFFFFFFFF), jnp.uint32(0x80000000))
    return uu ^ xm


def _make_select(n, k):
    cnt = n // _NT
    out_chunk = n // _NW
    cvecs = cnt // _LANES
    ovecs = out_chunk // _LANES

    mesh = plsc.VectorSubcoreMesh(
        core_axis_name="c", subcore_axis_name="s", num_cores=1)

    @functools.partial(
        pl.kernel,
        out_type=jax.ShapeDtypeStruct((n,), jnp.float32),
        mesh=mesh,
        compiler_params=pltpu.CompilerParams(needs_layout_passes=False),
        scratch_types=[
            pltpu.VMEM((cnt,), jnp.float32),
            pltpu.VMEM((cnt,), jnp.uint32),
            pltpu.VMEM((256,), jnp.int32),
            pltpu.VMEM((256,), jnp.int32),
            pltpu.VMEM((_NT, 256), jnp.int32),
            pltpu.VMEM_SHARED((2, _NT, 256), jnp.int32),
            pltpu.VMEM((cnt,), jnp.float32),
        ],
    )
    def select(scores_hbm, hist0_hbm, out_hbm, sc_v, keys_v, hist_v, g_v,
               readh_v, shared_h, outm_v):
        t = lax.axis_index("s")
        lane = lax.broadcasted_iota(jnp.int32, (_LANES,), 0)
        zero16 = jnp.zeros((_LANES,), jnp.int32)

        pltpu.sync_copy(scores_hbm.at[pl.ds(t * cnt, cnt)], sc_v)
        pltpu.sync_copy(hist0_hbm, g_v)

        def _kb(i, _):
            for u in range(8):
                off = (i * 8 + u) * _LANES
                keys_v[pl.ds(off, _LANES)] = _keys16(sc_v[pl.ds(off, _LANES)])
            return 0
        lax.fori_loop(0, cvecs // 8, _kb, 0)

        prefix = jnp.uint32(0)
        cglobal = jnp.int32(n)
        for r in range(4):
            shift = 24 - 8 * r
            if r > 0:
                for b in range(256 // _LANES):
                    hist_v[pl.ds(b * _LANES, _LANES)] = zero16

                def _hb(i, _):
                    for u in range(8):
                        off = (i * 8 + u) * _LANES
                        kv = keys_v[pl.ds(off, _LANES)]
                        byte = ((kv >> jnp.uint32(shift))
                                & jnp.uint32(255)).astype(jnp.int32)
                        active = (kv >> jnp.uint32(shift + 8)) == (
                            prefix >> jnp.uint32(shift + 8))
                        dcnt, lastm = plsc.scan_count(byte, active)
                        plsc.addupdate_scatter(
                            hist_v, [byte], dcnt, mask=lastm)
                    return 0
                lax.fori_loop(0, cvecs // 8, _hb, 0)

                parity = r % 2
                pltpu.sync_copy(hist_v, shared_h.at[parity, t])
                plsc.subcore_barrier()
                pltpu.sync_copy(shared_h.at[parity], readh_v)

                for b in range(256 // _LANES):
                    acc = readh_v[0, pl.ds(b * _LANES, _LANES)]
                    for tt in range(1, _NT):
                        acc = acc + readh_v[tt, pl.ds(b * _LANES, _LANES)]
                    g_v[pl.ds(b * _LANES, _LANES)] = acc

            tot_all = jnp.int32(0)
            for b in range(256 // _LANES):
                tot_all = tot_all + jnp.sum(g_v[pl.ds(b * _LANES, _LANES)])
            above = cglobal - tot_all
            run = above
            ncond = jnp.int32(0)
            cmin = jnp.int32(n)
            for b in range(256 // _LANES - 1, -1, -1):
                gv = g_v[pl.ds(b * _LANES, _LANES)]
                incl = plsc.cumsum(gv)
                tot_v = jnp.sum(gv)
                s_vec = run + (tot_v - incl + gv)
                cond = s_vec >= k
                ncond = ncond + jnp.sum(cond.astype(jnp.int32))
                cmin = jnp.minimum(
                    cmin, jnp.min(jnp.where(cond, s_vec, n)))
                run = run + tot_v
            jstar = ncond - 1
            prefix = prefix | (jstar.astype(jnp.uint32)
                               << jnp.uint32(shift))
            cglobal = cmin

        thr = prefix

        eq_total = jnp.int32(0)
        eq_before = jnp.int32(0)
        for b in range(256 // _LANES):
            sel_lane = lane == (jstar - b * _LANES)
            gv = g_v[pl.ds(b * _LANES, _LANES)]
            eq_total = eq_total + jnp.sum(jnp.where(sel_lane, gv, 0))
            pre = jnp.zeros((_LANES,), jnp.int32)
            for tt in range(_NT):
                row = readh_v[tt, pl.ds(b * _LANES, _LANES)]
                pre = pre + jnp.where(tt < t, row, 0)
            eq_before = eq_before + jnp.sum(jnp.where(sel_lane, pre, 0))
        gt_total = cglobal - eq_total
        needed = k - gt_total

        def _mb(i, carry):
            for u in range(8):
                off = (i * 8 + u) * _LANES
                ku = keys_v[pl.ds(off, _LANES)]
                eqm = ku == thr
                eqi = jnp.where(eqm, 1, 0)
                incl = plsc.cumsum(eqi)
                rank = carry + (incl - eqi)
                sel = (ku > thr) | (eqm & (rank < needed))
                outm_v[pl.ds(off, _LANES)] = jnp.where(sel, 1.0, 0.0)
                carry = carry + jnp.sum(eqi)
            return carry

        lax.fori_loop(0, cvecs // 8, _mb, eq_before)
        pltpu.sync_copy(outm_v, out_hbm.at[pl.ds(t * cnt, cnt)])

    return select


def kernel(x, W, b, step_embed, step):
    Bd, Sd, Hd = x.shape
    n = Bd * Sd
    k = max(1, int(round(n * 0.25)))

    x2 = x.reshape(n, Hd)
    bias = (b[0] + step_embed[step, 0]).reshape(1, 1).astype(jnp.float32)

    scores2, g2, hist0 = _compute_scores(x2, W.astype(jnp.float32), bias)
    scores_flat = scores2.reshape(n)

    m_flat = _make_select(n, k)(scores_flat, hist0.reshape(256))

    g = g2.reshape(Bd, Sd, 1)
    m = m_flat.reshape(Bd, Sd, 1)
    aux_loss = jnp.zeros((), x.dtype)
    return (g, m, aux_loss)


# speedup vs baseline: 10.4531x; 10.4531x over previous
import functools

import jax
import jax.numpy as jnp
from jax import lax
from jax.experimental import pallas as pl
from jax.experimental.pallas import tpu as pltpu
from jax.experimental.pallas import tpu_sc as plsc


def _scores_body(x_ref, w_ref, bias_ref, scores_ref, g_ref):
    xb = x_ref[...].astype(jnp.bfloat16).astype(jnp.float32)
    wv = w_ref[...].astype(jnp.bfloat16).astype(jnp.float32)
    s = jnp.sum(xb * wv, axis=-1)
    s = s + bias_ref[0, 0]
    scores_ref[0, 0, :] = s
    g_ref[0, 0, :] = jax.nn.sigmoid(s)


def _compute_scores(x2, W, bias):
    n, h = x2.shape
    rows = 2048
    grid = n // rows
    return pl.pallas_call(
        _scores_body,
        grid=(grid,),
        in_specs=[
            pl.BlockSpec((rows, h), lambda i: (i, 0)),
            pl.BlockSpec((1, h), lambda i: (0, 0)),
            pl.BlockSpec(memory_space=pltpu.SMEM),
        ],
        out_specs=[
            pl.BlockSpec((1, 1, rows), lambda i: (i, 0, 0)),
            pl.BlockSpec((1, 1, rows), lambda i: (i, 0, 0)),
        ],
        out_shape=[
            jax.ShapeDtypeStruct((grid, 1, rows), jnp.float32),
            jax.ShapeDtypeStruct((grid, 1, rows), jnp.float32),
        ],
        compiler_params=pltpu.CompilerParams(
            dimension_semantics=("arbitrary",),
        ),
    )(x2, W, bias)


_LANES = 16
_NT = 16
_NW = 32


def _keys16(v):
    uu = lax.bitcast_convert_type(v, jnp.uint32)
    neg = uu >= jnp.uint32(0x80000000)
    xm = jnp.where(neg, jnp.uint32(0xFFFFFFFF), jnp.uint32(0x80000000))
    return uu ^ xm


def _make_select(n, k):
    cnt = n // _NT
    out_chunk = n // _NW
    cvecs = cnt // _LANES
    ovecs = out_chunk // _LANES

    mesh = plsc.VectorSubcoreMesh(
        core_axis_name="c", subcore_axis_name="s", num_cores=1)

    @functools.partial(
        pl.kernel,
        out_type=jax.ShapeDtypeStruct((n,), jnp.float32),
        mesh=mesh,
        compiler_params=pltpu.CompilerParams(needs_layout_passes=False),
        scratch_types=[
            pltpu.VMEM((cnt,), jnp.float32),
            pltpu.VMEM((cnt,), jnp.uint32),
            pltpu.VMEM((256,), jnp.int32),
            pltpu.VMEM((256,), jnp.int32),
            pltpu.VMEM((_NT, 256), jnp.int32),
            pltpu.VMEM_SHARED((2, _NT, 256), jnp.int32),
            pltpu.VMEM((cnt,), jnp.float32),
        ],
    )
    def select(scores_hbm, out_hbm, sc_v, keys_v, hist_v, g_v,
               readh_v, shared_h, outm_v):
        t = lax.axis_index("s")
        lane = lax.broadcasted_iota(jnp.int32, (_LANES,), 0)
        zero16 = jnp.zeros((_LANES,), jnp.int32)

        pltpu.sync_copy(scores_hbm.at[pl.ds(t * cnt, cnt)], sc_v)

        def _kb(i, _):
            for u in range(8):
                off = (i * 8 + u) * _LANES
                keys_v[pl.ds(off, _LANES)] = _keys16(sc_v[pl.ds(off, _LANES)])
            return 0
        lax.fori_loop(0, cvecs // 8, _kb, 0)

        prefix = jnp.uint32(0)
        cglobal = jnp.int32(n)
        for r in range(4):
            shift = 24 - 8 * r
            for b in range(256 // _LANES):
                hist_v[pl.ds(b * _LANES, _LANES)] = zero16

            def _hb(i, _):
                for u in range(8):
                    off = (i * 8 + u) * _LANES
                    kv = keys_v[pl.ds(off, _LANES)]
                    byte = ((kv >> jnp.uint32(shift))
                            & jnp.uint32(255)).astype(jnp.int32)
                    if r == 0:
                        dcnt, lastm = plsc.scan_count(byte)
                    else:
                        active = (kv >> jnp.uint32(shift + 8)) == (
                            prefix >> jnp.uint32(shift + 8))
                        dcnt, lastm = plsc.scan_count(byte, active)
                    plsc.addupdate_scatter(hist_v, [byte], dcnt, mask=lastm)
                return 0
            lax.fori_loop(0, cvecs // 8, _hb, 0)

            parity = r % 2
            pltpu.sync_copy(hist_v, shared_h.at[parity, t])
            plsc.subcore_barrier()
            pltpu.sync_copy(shared_h.at[parity], readh_v)

            for b in range(256 // _LANES):
                acc = readh_v[0, pl.ds(b * _LANES, _LANES)]
                for tt in range(1, _NT):
                    acc = acc + readh_v[tt, pl.ds(b * _LANES, _LANES)]
                g_v[pl.ds(b * _LANES, _LANES)] = acc

            tot_all = jnp.int32(0)
            for b in range(256 // _LANES):
                tot_all = tot_all + jnp.sum(g_v[pl.ds(b * _LANES, _LANES)])
            above = cglobal - tot_all
            run = above
            ncond = jnp.int32(0)
            cmin = jnp.int32(n)
            for b in range(256 // _LANES - 1, -1, -1):
                gv = g_v[pl.ds(b * _LANES, _LANES)]
                incl = plsc.cumsum(gv)
                tot_v = jnp.sum(gv)
                s_vec = run + (tot_v - incl + gv)
                cond = s_vec >= k
                ncond = ncond + jnp.sum(cond.astype(jnp.int32))
                cmin = jnp.minimum(
                    cmin, jnp.min(jnp.where(cond, s_vec, n)))
                run = run + tot_v
            jstar = ncond - 1
            prefix = prefix | (jstar.astype(jnp.uint32)
                               << jnp.uint32(shift))
            cglobal = cmin

        thr = prefix

        eq_total = jnp.int32(0)
        eq_before = jnp.int32(0)
        for b in range(256 // _LANES):
            sel_lane = lane == (jstar - b * _LANES)
            gv = g_v[pl.ds(b * _LANES, _LANES)]
            eq_total = eq_total + jnp.sum(jnp.where(sel_lane, gv, 0))
            pre = jnp.zeros((_LANES,), jnp.int32)
            for tt in range(_NT):
                row = readh_v[tt, pl.ds(b * _LANES, _LANES)]
                pre = pre + jnp.where(tt < t, row, 0)
            eq_before = eq_before + jnp.sum(jnp.where(sel_lane, pre, 0))
        gt_total = cglobal - eq_total
        needed = k - gt_total

        def _mb(i, carry):
            for u in range(8):
                off = (i * 8 + u) * _LANES
                ku = keys_v[pl.ds(off, _LANES)]
                eqm = ku == thr
                eqi = jnp.where(eqm, 1, 0)
                incl = plsc.cumsum(eqi)
                rank = carry + (incl - eqi)
                sel = (ku > thr) | (eqm & (rank < needed))
                outm_v[pl.ds(off, _LANES)] = jnp.where(sel, 1.0, 0.0)
                carry = carry + jnp.sum(eqi)
            return carry

        lax.fori_loop(0, cvecs // 8, _mb, eq_before)
        pltpu.sync_copy(outm_v, out_hbm.at[pl.ds(t * cnt, cnt)])

    return select


def kernel(x, W, b, step_embed, step):
    Bd, Sd, Hd = x.shape
    n = Bd * Sd
    k = max(1, int(round(n * 0.25)))

    x2 = x.reshape(n, Hd)
    bias = (b[0] + step_embed[step, 0]).reshape(1, 1).astype(jnp.float32)

    scores2, g2 = _compute_scores(x2, W.astype(jnp.float32), bias)
    scores_flat = scores2.reshape(n)

    m_flat = _make_select(n, k)(scores_flat)

    g = g2.reshape(Bd, Sd, 1)
    m = m_flat.reshape(Bd, Sd, 1)
    aux_loss = jnp.zeros((), x.dtype)
    return (g, m, aux_loss)
